# Initial kernel scaffold; baseline (speedup 1.0000x reference)
import jax
import jax.numpy as jnp
from jax.experimental import pallas as pl


def kernel(atom_W, in_W, out_W, graph_token_W, type_W, x, in_degree, out_degree):
    raise NotImplementedError("write your pallas kernel here")


# speedup vs baseline: 4.4099x; 4.4099x over previous
import functools

import jax
import jax.numpy as jnp
from jax import lax
from jax.experimental import pallas as pl
from jax.experimental.pallas import tpu as pltpu
from jax.experimental.pallas import tpu_sc as plsc

C = 4
K = 11
L = 16


@functools.lru_cache(maxsize=None)
def _build_sc_fn(B, N, F, H, T):
    info = plsc.get_sparse_core_info()
    NW = info.num_cores * info.num_subcores
    nodes = B * N
    assert nodes % NW == 0
    npw = nodes // NW
    assert npw % C == 0 and N % C == 0
    S = npw // C
    assert (npw % N) == 0
    bpw = npw // N
    KC = K * C
    HL = H // L
    assert H % L == 0

    mesh = plsc.VectorSubcoreMesh(core_axis_name="c", subcore_axis_name="s")

    def body(table_hbm, idx_hbm, gtok_hbm, type_hbm, out_hbm,
             idx_v, g0, g1, o0, o1, type_v, gtok_v, tok_v,
             gs0, gs1, ws0, ws1):
        wid = lax.axis_index("s") * info.num_cores + lax.axis_index("c")
        step0 = wid * S

        def gather(s, buf, sem):
            return pltpu.make_async_copy(table_hbm.at[idx_v.at[s]], buf, sem)

        def write(s, buf, sem):
            gn = step0 * C + s * C
            b = gn // N
            row = b * (N + 1) + (gn % N) + 1
            return pltpu.make_async_copy(buf, out_hbm.at[pl.ds(row, C)], sem)

        pltpu.sync_copy(idx_hbm.at[pl.ds(step0, S)], idx_v)
        gather(0, g0, gs0).start()
        pltpu.sync_copy(type_hbm.at[0], type_v)
        pltpu.sync_copy(gtok_hbm.at[0], gtok_v)

        def tok_max(j, m):
            return jnp.maximum(m, jnp.abs(gtok_v[pl.ds(j * L, L)]))
        t = jnp.where(jnp.max(lax.fori_loop(0, HL, tok_max,
                                            jnp.zeros((L,), jnp.float32))) > 0.0,
                      1.0, 0.0).astype(jnp.float32)

        def tok_fill(j, _):
            sl = pl.ds(j * L, L)
            tok_v[sl] = gtok_v[sl] + t * type_v[sl]
            return 0
        lax.fori_loop(0, HL, tok_fill, 0)

        def tok_write(i, _):
            b = wid * bpw + i
            pltpu.sync_copy(tok_v, out_hbm.at[b * (N + 1)])
            return 0
        lax.fori_loop(0, bpw, tok_write, 0)

        def compute(buf, obuf):
            for n in range(C):
                def col_sum(j, m):
                    sl = pl.ds(j * L, L)
                    acc = buf[n * K, sl]
                    for r in range(1, K):
                        acc = acc + buf[n * K + r, sl]
                    obuf[n, sl] = acc
                    return jnp.maximum(m, jnp.abs(acc))
                mx = lax.fori_loop(0, HL, col_sum, jnp.zeros((L,), jnp.float32))
                tn = jnp.where(jnp.max(mx) > 0.0, 1.0, 0.0).astype(jnp.float32)

                def col_fix(j, _):
                    sl = pl.ds(j * L, L)
                    obuf[n, sl] = obuf[n, sl] + tn * type_v[sl]
                    return 0
                lax.fori_loop(0, HL, col_fix, 0)

        gbuf = (g0, g1)
        gsem = (gs0, gs1)
        obuf = (o0, o1)
        wsem = (ws0, ws1)

        def step_pair(s2, _):
            for p in range(2):
                s = s2 * 2 + p

                @pl.when(s + 1 < S)
                def _():
                    gather(s + 1, gbuf[1 - p], gsem[1 - p]).start()

                gather(s, gbuf[p], gsem[p]).wait()

                @pl.when(s >= 2)
                def _():
                    write(s - 2, obuf[p], wsem[p]).wait()

                compute(gbuf[p], obuf[p])
                write(s, obuf[p], wsem[p]).start()
            return 0

        lax.fori_loop(0, S // 2, step_pair, 0)
        write(S - 2, obuf[0], wsem[0]).wait()
        write(S - 1, obuf[1], wsem[1]).wait()

    fn = pl.kernel(
        body,
        out_type=jax.ShapeDtypeStruct((B * (N + 1), H), jnp.float32),
        mesh=mesh,
        scratch_types=[
            pltpu.VMEM((S, KC), jnp.int32),
            pltpu.VMEM((KC, H), jnp.float32),
            pltpu.VMEM((KC, H), jnp.float32),
            pltpu.VMEM((C, H), jnp.float32),
            pltpu.VMEM((C, H), jnp.float32),
            pltpu.VMEM((H,), jnp.float32),
            pltpu.VMEM((H,), jnp.float32),
            pltpu.VMEM((H,), jnp.float32),
            pltpu.SemaphoreType.DMA,
            pltpu.SemaphoreType.DMA,
            pltpu.SemaphoreType.DMA,
            pltpu.SemaphoreType.DMA,
        ],
        compiler_params=pltpu.CompilerParams(use_tc_tiling_on_sc=False,
                                             needs_layout_passes=False),
    )
    return fn


def kernel(atom_W, in_W, out_W, graph_token_W, type_W, x, in_degree, out_degree):
    B, N, F = x.shape
    H = atom_W.shape[-1]
    NA = atom_W.shape[0]
    NI = in_W.shape[0]
    NO = out_W.shape[0]

    table = jnp.concatenate(
        [atom_W.at[0].set(0.0), in_W.at[0].set(0.0), out_W.at[0].set(0.0)],
        axis=0).astype(jnp.float32)
    idx = jnp.concatenate(
        [x.astype(jnp.int32),
         in_degree.astype(jnp.int32)[..., None] + jnp.int32(NA),
         out_degree.astype(jnp.int32)[..., None] + jnp.int32(NA + NI)],
        axis=-1)
    idx2 = idx.reshape(B * N // C, K * C)

    assert K == F + 2
    fn = _build_sc_fn(B, N, F, H, NA + NI + NO)
    out = fn(table, idx2,
             graph_token_W.astype(jnp.float32), type_W.astype(jnp.float32))
    return out.reshape(B, N + 1, H)
